# Initial kernel scaffold; baseline (speedup 1.0000x reference)
import jax
import jax.numpy as jnp
from jax.experimental import pallas as pl


def kernel(inputs, anchors):
    raise NotImplementedError("write your pallas kernel here")


# speedup vs baseline: 27.4671x; 27.4671x over previous
import jax
import jax.numpy as jnp
from jax.experimental import pallas as pl
from jax.experimental.pallas import tpu as pltpu

GRID = 64
NB = 5
NC = 20
SIZE = 512
DET_TH = 0.1
NMS_TH = 0.5
MAX_BOXES = 50
N = GRID * GRID * NB
RR, LL = 160, 128
SLOT = 64
J = NC * SLOT

_NEG_INF = float("-inf")


def _decode_body(x_ref, anch_ref, x1_ref, y1_ref, x2_ref, y2_ref, s_ref):
    row = jax.lax.broadcasted_iota(jnp.int32, (RR, LL), 0)
    lane = jax.lax.broadcasted_iota(jnp.int32, (RR, LL), 1)
    flat = row * LL + lane
    a = flat % NB
    gidx = flat // NB
    gx = (gidx % GRID).astype(jnp.float32)
    gy = (gidx // GRID).astype(jnp.float32)

    aw = jnp.zeros((RR, LL), jnp.float32)
    ah = jnp.zeros((RR, LL), jnp.float32)
    for k in range(NB):
        aw = jnp.where(a == k, anch_ref[k, 0], aw)
        ah = jnp.where(a == k, anch_ref[k, 1], ah)

    bx = (jax.nn.sigmoid(x_ref[0]) + gx) / GRID
    by = (jax.nn.sigmoid(x_ref[1]) + gy) / GRID
    bw = jnp.exp(x_ref[2]) * aw / GRID
    bh = jnp.exp(x_ref[3]) * ah / GRID
    x1_ref[...] = bx - bw / 2.0
    y1_ref[...] = by - bh / 2.0
    x2_ref[...] = bx + bw / 2.0
    y2_ref[...] = by + bh / 2.0

    conf = jax.nn.sigmoid(x_ref[4])
    cls = [x_ref[5 + c] for c in range(NC)]
    m = cls[0]
    for c in range(1, NC):
        m = jnp.maximum(m, cls[c])
    es = [jnp.exp(cls[c] - m) for c in range(NC)]
    ssum = es[0]
    for c in range(1, NC):
        ssum = ssum + es[c]
    th = []
    for c in range(NC):
        sc = conf * (es[c] / ssum)
        th.append(sc * (sc > DET_TH).astype(jnp.float32))
    best = th[0]
    bcls = jnp.zeros((RR, LL), jnp.int32)
    for c in range(1, NC):
        gt = th[c] > best
        best = jnp.where(gt, th[c], best)
        bcls = jnp.where(gt, c, bcls)
    pos = best > 0.0
    for c in range(NC):
        s_ref[c] = jnp.where((bcls == c) & pos, best, 0.0)


def _nms_body(x1_ref, y1_ref, x2_ref, y2_ref, s_in_ref,
              ox1_ref, oy1_ref, ox2_ref, oy2_ref, s_ref):
    s_ref[...] = s_in_ref[...]
    x1v = x1_ref[...]
    y1v = y1_ref[...]
    x2v = x2_ref[...]
    y2v = y2_ref[...]
    a2v = jnp.maximum(x2v - x1v, 0.0) * jnp.maximum(y2v - y1v, 0.0)
    iota = (jax.lax.broadcasted_iota(jnp.int32, (RR, LL), 0) * LL
            + jax.lax.broadcasted_iota(jnp.int32, (RR, LL), 1))
    rowio = jax.lax.broadcasted_iota(jnp.int32, (NC, SLOT), 0)
    laneio = jax.lax.broadcasted_iota(jnp.int32, (NC, SLOT), 1)
    big = jnp.int32(2**30)

    def step(t, carry):
        sx1, sy1, sx2, sy2 = carry
        for c in range(NC):
            s = s_ref[c]
            mval = jnp.max(s)
            idx = jnp.min(jnp.where(s == mval, iota, big))
            em = iota == idx
            bx1 = jnp.sum(jnp.where(em, x1v, 0.0))
            by1 = jnp.sum(jnp.where(em, y1v, 0.0))
            bx2 = jnp.sum(jnp.where(em, x2v, 0.0))
            by2 = jnp.sum(jnp.where(em, y2v, 0.0))
            yy1 = jnp.maximum(bx1, x1v)
            xx1 = jnp.maximum(by1, y1v)
            yy2 = jnp.minimum(bx2, x2v)
            xx2 = jnp.minimum(by2, y2v)
            inter = jnp.maximum(yy2 - yy1, 0.0) * jnp.maximum(xx2 - xx1, 0.0)
            a1 = (jnp.maximum(bx2 - bx1, 0.0) * jnp.maximum(by2 - by1, 0.0))
            iou = inter / (a1 + a2v - inter + 1e-9)
            s_ref[c] = jnp.where((iou > NMS_TH) | em, _NEG_INF, s)
            upd = (rowio == c) & (laneio == t)
            sx1 = jnp.where(upd, bx1, sx1)
            sy1 = jnp.where(upd, by1, sy1)
            sx2 = jnp.where(upd, bx2, sx2)
            sy2 = jnp.where(upd, by2, sy2)
        return sx1, sy1, sx2, sy2

    zero = jnp.zeros((NC, SLOT), jnp.float32)
    sx1, sy1, sx2, sy2 = jax.lax.fori_loop(
        0, MAX_BOXES, step, (zero, zero, zero, zero))
    ox1_ref[...] = sx1
    oy1_ref[...] = sy1
    ox2_ref[...] = sx2
    oy2_ref[...] = sy2


def _raster_body(c0_ref, c1_ref, c2_ref, c3_ref, out_ref):
    py1 = jnp.clip(jnp.floor(c0_ref[...] * 512.0), 0.0, 511.0)
    px1 = jnp.clip(jnp.floor(c1_ref[...] * 512.0), 0.0, 511.0)
    py2 = jnp.clip(jnp.floor(c2_ref[...] * 512.0), 0.0, 511.0)
    px2 = jnp.clip(jnp.floor(c3_ref[...] * 512.0), 0.0, 511.0)
    pix = jax.lax.broadcasted_iota(jnp.int32, (J, SIZE), 1).astype(jnp.float32)
    slotio = jax.lax.broadcasted_iota(jnp.int32, (J, SIZE), 0) % SLOT
    valid = slotio < MAX_BOXES
    hh = (((pix == py1) | (pix == py2)) & valid).astype(jnp.bfloat16)
    rr = ((pix >= px1) & (pix <= px2)).astype(jnp.bfloat16)
    vv = (((pix == px1) | (pix == px2)) & valid).astype(jnp.bfloat16)
    cc = ((pix >= py1) & (pix <= py2)).astype(jnp.bfloat16)
    dn = (((0,), (0,)), ((), ()))
    cnt = (jax.lax.dot_general(hh, rr, dn, preferred_element_type=jnp.float32)
           + jax.lax.dot_general(cc, vv, dn, preferred_element_type=jnp.float32))
    out_ref[...] = (cnt > 0.0).astype(jnp.float32)


def kernel(inputs, anchors):
    xt = inputs.reshape(N, 5 + NC).T.reshape(5 + NC, RR, LL)

    f = jax.ShapeDtypeStruct
    x1, y1, x2, y2, s = pl.pallas_call(
        _decode_body,
        in_specs=[pl.BlockSpec(memory_space=pltpu.VMEM),
                  pl.BlockSpec(memory_space=pltpu.SMEM)],
        out_shape=[f((RR, LL), jnp.float32)] * 4 + [f((NC, RR, LL), jnp.float32)],
    )(xt, anchors)

    sx1, sy1, sx2, sy2 = pl.pallas_call(
        _nms_body,
        out_shape=[f((NC, SLOT), jnp.float32)] * 4,
        scratch_shapes=[pltpu.VMEM((NC, RR, LL), jnp.float32)],
    )(x1, y1, x2, y2, s)

    mask = pl.pallas_call(
        _raster_body,
        out_shape=f((SIZE, SIZE), jnp.float32),
    )(sx1.reshape(J, 1), sy1.reshape(J, 1),
      sx2.reshape(J, 1), sy2.reshape(J, 1))
    return mask.reshape(1, SIZE, SIZE, 1)
